# baseline (speedup 1.0000x reference)
import jax
import jax.numpy as jnp
from jax import lax
from jax.experimental import pallas as pl
from jax.experimental.pallas import tpu as pltpu
from jax.experimental.pallas import tpu_sc as plsc

N = 10000
D_IN = 256
HID = 128
HEADS = 8
E = 160000
OUT = 128

BLK = 1000
NT = 16
CH = 64
NCHP = 157
EA = (NCHP + 1) * NT * CH
NPAD = N + 8
RCH = 40
NRCH = N // RCH


def _proj_body(xg_ref, xd_ref, wg_ref, bg_ref, wd_ref, bd_ref, mg_ref, md_ref,
               hg_ref, hd_ref, ag_ref, ad_ref):
    hg = jnp.dot(xg_ref[...], wg_ref[...], preferred_element_type=jnp.float32) + bg_ref[...]
    hd = jnp.dot(xd_ref[...], wd_ref[...], preferred_element_type=jnp.float32) + bd_ref[...]
    hg_ref[...] = hg
    hd_ref[...] = hd
    ag_ref[...] = jnp.dot(hg, mg_ref[...], preferred_element_type=jnp.float32)
    ad_ref[...] = jnp.dot(hd, md_ref[...], preferred_element_type=jnp.float32)


def _proj_call(xg, xd, wg, bg, wd, bd, mg, md):
    rb = lambda i: (i, 0)
    full = lambda i: (0, 0)
    return pl.pallas_call(
        _proj_body,
        grid=(N // BLK,),
        in_specs=[
            pl.BlockSpec((BLK, D_IN), rb),
            pl.BlockSpec((BLK, D_IN), rb),
            pl.BlockSpec((D_IN, HID), full),
            pl.BlockSpec((1, HID), full),
            pl.BlockSpec((D_IN, HID), full),
            pl.BlockSpec((1, HID), full),
            pl.BlockSpec((HID, 64), full),
            pl.BlockSpec((HID, 64), full),
        ],
        out_specs=[
            pl.BlockSpec((BLK, HID), rb),
            pl.BlockSpec((BLK, HID), rb),
            pl.BlockSpec((BLK, 64), rb),
            pl.BlockSpec((BLK, 64), rb),
        ],
        out_shape=[
            jax.ShapeDtypeStruct((N, HID), jnp.float32),
            jax.ShapeDtypeStruct((N, HID), jnp.float32),
            jax.ShapeDtypeStruct((N, 64), jnp.float32),
            jax.ShapeDtypeStruct((N, 64), jnp.float32),
        ],
    )(xg, xd, wg, bg, wd, bd, mg, md)


def _nchunks_t(t, total):
    return jnp.where(t < total - NT * (total // NT), total // NT + 1, total // NT)


def _sc_w_body(atab_hbm, src_hbm, dst_hbm,
               w_out_hbm, ssum_hbm,
               s_sh, idx2, idx_d, abrows, wpack, sbuf, zbuf,
               sem_a0, sem_a1, sem_b0, sem_b1, sem_g0, sem_g1):
    sem_a = (sem_a0, sem_a1)
    sem_b = (sem_b0, sem_b1)
    sem_g = (sem_g0, sem_g1)
    c = lax.axis_index("c")
    t = lax.axis_index("s")

    def zrow(i, _):
        for j in range(HID // 16):
            zbuf[i, pl.ds(16 * j, 16)] = jnp.zeros((16,), jnp.float32)
        return 0
    lax.fori_loop(0, RCH, zrow, 0)

    def zsrow(i, _):
        for j in range(HID // 16):
            sbuf[i, pl.ds(16 * j, 16)] = jnp.zeros((16,), jnp.float32)
        return 0
    lax.fori_loop(0, CH, zsrow, 0)

    def zchunk(k, _):
        r = (k * NT + t) * RCH
        pltpu.sync_copy(zbuf, s_sh.at[pl.ds(r, RCH)])
        return 0
    lax.fori_loop(0, _nchunks_t(t, NRCH), zchunk, 0)

    plsc.subcore_barrier()

    def start_idx(k, sl):
        b = c * EA + (k * NT + t) * CH
        pltpu.async_copy(src_hbm.at[pl.ds(b, CH)], idx2.at[sl, pl.ds(0, CH)],
                         sem_a[sl])
        pltpu.async_copy(dst_hbm.at[pl.ds(b, CH)], idx2.at[sl, pl.ds(CH, CH)],
                         sem_a[sl])
        pltpu.async_copy(dst_hbm.at[pl.ds(b, CH)], idx_d.at[sl], sem_b[sl])

    def start_gather(k, sl):
        b = c * EA + (k * NT + t) * CH
        pltpu.make_async_copy(src_hbm.at[pl.ds(b, CH)], idx2.at[sl, pl.ds(0, CH)],
                              sem_a[sl]).wait()
        pltpu.make_async_copy(dst_hbm.at[pl.ds(b, CH)], idx2.at[sl, pl.ds(CH, CH)],
                              sem_a[sl]).wait()
        pltpu.async_copy(atab_hbm.at[c].at[idx2.at[sl]], abrows.at[sl],
                         sem_g[sl])

    def process(k, sl):
        b = c * EA + (k * NT + t) * CH
        pltpu.make_async_copy(atab_hbm.at[c].at[idx2.at[sl]], abrows.at[sl],
                              sem_g[sl]).wait()
        pltpu.make_async_copy(dst_hbm.at[pl.ds(b, CH)], idx_d.at[sl],
                              sem_b[sl]).wait()

        @plsc.parallel_loop(0, CH, unroll=8)
        def wrow(i):
            a16 = abrows[sl, i, pl.ds(0, 16)]
            b16 = abrows[sl, CH + i, pl.ds(8, 16)]
            x = a16 + b16
            x = jnp.where(x >= 0.0, x, 0.2 * x)
            w = jnp.exp(x)
            sbuf[i, pl.ds(0, 16)] = w
            wpack[i // 8, pl.ds((i % 8) * 16, 16)] = w

        pltpu.sync_copy(sbuf, s_sh.at[idx_d.at[sl]], add=True)
        pltpu.sync_copy(wpack, w_out_hbm.at[pl.ds(pl.multiple_of(b // 8, 8), CH // 8)])

    start_idx(0, 0)
    start_gather(0, 0)
    start_idx(1, 1)

    def pair(kk, _):
        k0 = 2 * kk
        start_gather(k0 + 1, 1)
        process(k0, 0)
        start_idx(k0 + 2, 0)
        start_gather(k0 + 2, 0)
        process(k0 + 1, 1)
        start_idx(k0 + 3, 1)
        return 0
    lax.fori_loop(0, (NCHP - 1) // 2, pair, 0)
    process(NCHP - 1, 0)
    pltpu.make_async_copy(src_hbm.at[pl.ds(0, CH)], idx2.at[1, pl.ds(0, CH)],
                          sem_a[1]).wait()
    pltpu.make_async_copy(dst_hbm.at[pl.ds(0, CH)], idx2.at[1, pl.ds(CH, CH)],
                          sem_a[1]).wait()
    pltpu.make_async_copy(dst_hbm.at[pl.ds(0, CH)], idx_d.at[1], sem_b[1]).wait()

    plsc.subcore_barrier()

    def wb(k, _):
        r = (k * NT + t) * RCH
        pltpu.sync_copy(s_sh.at[pl.ds(r, RCH)], zbuf)
        pltpu.sync_copy(zbuf, ssum_hbm.at[c, pl.ds(r, RCH)])
        return 0
    lax.fori_loop(0, _nchunks_t(t, NRCH), wb, 0)


def _sc_m_body(h_hbm, w_hbm, src_hbm, dst_hbm,
               acc_hbm,
               acc_sh, idx_s, idx_d, wpack, hrows, zbuf,
               sem_a0, sem_a1, sem_b0, sem_b1, sem_g0, sem_g1):
    sem_a = (sem_a0, sem_a1)
    sem_b = (sem_b0, sem_b1)
    sem_g = (sem_g0, sem_g1)
    c = lax.axis_index("c")
    t = lax.axis_index("s")

    def zrow(i, _):
        for j in range(HID // 16):
            zbuf[i, pl.ds(16 * j, 16)] = jnp.zeros((16,), jnp.float32)
        return 0
    lax.fori_loop(0, RCH, zrow, 0)

    def zchunk(k, _):
        r = (k * NT + t) * RCH
        pltpu.sync_copy(zbuf, acc_sh.at[pl.ds(r, RCH)])
        return 0
    lax.fori_loop(0, _nchunks_t(t, NRCH), zchunk, 0)

    plsc.subcore_barrier()

    def start_idx(k, sl):
        b = c * EA + (k * NT + t) * CH
        pltpu.async_copy(src_hbm.at[pl.ds(b, CH)], idx_s.at[sl], sem_a[sl])
        pltpu.async_copy(dst_hbm.at[pl.ds(b, CH)], idx_d.at[sl], sem_b[sl])
        pltpu.async_copy(
            w_hbm.at[pl.ds(pl.multiple_of(b // 8, 8), CH // 8)], wpack.at[sl],
            sem_b[sl])

    def start_gather(k, sl):
        b = c * EA + (k * NT + t) * CH
        pltpu.make_async_copy(src_hbm.at[pl.ds(b, CH)], idx_s.at[sl],
                              sem_a[sl]).wait()
        pltpu.async_copy(h_hbm.at[c].at[idx_s.at[sl]], hrows.at[sl],
                         sem_g[sl])

    def process(k, sl):
        b = c * EA + (k * NT + t) * CH
        pltpu.make_async_copy(dst_hbm.at[pl.ds(b, CH)], idx_d.at[sl],
                              sem_b[sl]).wait()
        pltpu.make_async_copy(
            w_hbm.at[pl.ds(pl.multiple_of(b // 8, 8), CH // 8)], wpack.at[sl],
            sem_b[sl]).wait()
        pltpu.make_async_copy(h_hbm.at[c].at[idx_s.at[sl]], hrows.at[sl],
                              sem_g[sl]).wait()

        @plsc.parallel_loop(0, CH, unroll=8)
        def srow(i):
            wrow = wpack[sl, i // 8, pl.ds((i % 8) * 16, 16)]
            for h in range(HEADS):
                hrows[sl, i, pl.ds(16 * h, 16)] = (
                    hrows[sl, i, pl.ds(16 * h, 16)] * wrow[h])

        pltpu.sync_copy(hrows.at[sl], acc_sh.at[idx_d.at[sl]], add=True)

    start_idx(0, 0)
    start_gather(0, 0)
    start_idx(1, 1)

    def pair(kk, _):
        k0 = 2 * kk
        start_gather(k0 + 1, 1)
        process(k0, 0)
        start_idx(k0 + 2, 0)
        start_gather(k0 + 2, 0)
        process(k0 + 1, 1)
        start_idx(k0 + 3, 1)
        return 0
    lax.fori_loop(0, (NCHP - 1) // 2, pair, 0)
    process(NCHP - 1, 0)
    pltpu.make_async_copy(src_hbm.at[pl.ds(0, CH)], idx_s.at[1], sem_a[1]).wait()
    pltpu.make_async_copy(dst_hbm.at[pl.ds(0, CH)], idx_d.at[1], sem_b[1]).wait()
    pltpu.make_async_copy(w_hbm.at[pl.ds(0, CH // 8)], wpack.at[1], sem_b[1]).wait()

    plsc.subcore_barrier()

    def wb(k, _):
        r = (k * NT + t) * RCH
        pltpu.sync_copy(acc_sh.at[pl.ds(r, RCH)], zbuf)
        pltpu.sync_copy(zbuf, acc_hbm.at[c, pl.ds(r, RCH)])
        return 0
    lax.fori_loop(0, _nchunks_t(t, NRCH), wb, 0)


def _sc_call(h_stack, atab, src_stack, dst_stack):
    mesh = plsc.VectorSubcoreMesh(core_axis_name="c", subcore_axis_name="s")
    w_all, ssum = pl.kernel(
        _sc_w_body,
        out_type=[
            jax.ShapeDtypeStruct((2 * EA // 8, HID), jnp.float32),
            jax.ShapeDtypeStruct((2, N, HID), jnp.float32),
        ],
        mesh=mesh,
        scratch_types=[
            pltpu.VMEM_SHARED((NPAD, HID), jnp.float32),
            pltpu.VMEM((2, 2 * CH), jnp.int32),
            pltpu.VMEM((2, CH), jnp.int32),
            pltpu.VMEM((2, 2 * CH, HID), jnp.float32),
            pltpu.VMEM((CH // 8, HID), jnp.float32),
            pltpu.VMEM((CH, HID), jnp.float32),
            pltpu.VMEM((RCH, HID), jnp.float32),
        ] + [pltpu.SemaphoreType.DMA] * 6,
    )(atab, src_stack, dst_stack)

    acc = pl.kernel(
        _sc_m_body,
        out_type=jax.ShapeDtypeStruct((2, N, HID), jnp.float32),
        mesh=plsc.VectorSubcoreMesh(core_axis_name="c", subcore_axis_name="s"),
        scratch_types=[
            pltpu.VMEM_SHARED((NPAD, HID), jnp.float32),
            pltpu.VMEM((2, CH), jnp.int32),
            pltpu.VMEM((2, CH), jnp.int32),
            pltpu.VMEM((2, CH // 8, HID), jnp.float32),
            pltpu.VMEM((2, CH, HID), jnp.float32),
            pltpu.VMEM((RCH, HID), jnp.float32),
        ] + [pltpu.SemaphoreType.DMA] * 6,
    )(h_stack, w_all, src_stack, dst_stack)
    return acc, ssum


def _norm_body(acc_ref, s_ref, r_ref, kw_ref, kb_ref, outn_ref, tsum_ref):
    t = pl.program_id(0)
    i = pl.program_id(1)
    srep = jnp.dot(s_ref[0], r_ref[...], preferred_element_type=jnp.float32)
    a = acc_ref[0]
    o = jnp.where(srep > 0.0, a / srep, 0.0)
    o = jnp.maximum(o, 0.0)
    outn_ref[0] = o
    ts = jnp.sum(
        jnp.tanh(jnp.dot(o, kw_ref[...], preferred_element_type=jnp.float32) + kb_ref[...]),
        axis=0, keepdims=True)

    @pl.when((t == 0) & (i == 0))
    def _():
        tsum_ref[...] = jnp.zeros((2, HID), jnp.float32)

    row = lax.broadcasted_iota(jnp.int32, (2, HID), 0)
    tsum_ref[...] = tsum_ref[...] + jnp.where(row == t, ts, 0.0)


def _norm_call(acc, ssum, r, kw, kb):
    tb = lambda t, i: (t, i, 0)
    full = lambda t, i: (0, 0)
    return pl.pallas_call(
        _norm_body,
        grid=(2, N // BLK),
        in_specs=[
            pl.BlockSpec((1, BLK, HID), tb),
            pl.BlockSpec((1, BLK, HID), tb),
            pl.BlockSpec((HID, HID), full),
            pl.BlockSpec((HID, HID), full),
            pl.BlockSpec((1, HID), full),
        ],
        out_specs=[
            pl.BlockSpec((1, BLK, HID), tb),
            pl.BlockSpec((2, HID), lambda t, i: (0, 0)),
        ],
        out_shape=[
            jax.ShapeDtypeStruct((2, N, HID), jnp.float32),
            jax.ShapeDtypeStruct((2, HID), jnp.float32),
        ],
    )(acc, ssum, r, kw, kb)


def _fin_body(attn_ref, outn_ref, w_ref, b_ref, o_ref):
    g = attn_ref[0] * outn_ref[0] + attn_ref[1] * outn_ref[1]
    o_ref[...] = jnp.dot(g, w_ref[...], preferred_element_type=jnp.float32) + b_ref[...]


def _fin_call(attn, outn, w, b):
    return pl.pallas_call(
        _fin_body,
        grid=(N // BLK,),
        in_specs=[
            pl.BlockSpec(memory_space=pltpu.SMEM),
            pl.BlockSpec((2, BLK, HID), lambda i: (0, i, 0)),
            pl.BlockSpec((HID, OUT), lambda i: (0, 0)),
            pl.BlockSpec((1, OUT), lambda i: (0, 0)),
        ],
        out_specs=pl.BlockSpec((BLK, OUT), lambda i: (i, 0)),
        out_shape=jax.ShapeDtypeStruct((N, OUT), jnp.float32),
    )(attn, outn, w, b)


def _att_block(att):
    eye8 = jnp.eye(HEADS, dtype=jnp.float32)
    b = (att[0][:, :, None] * eye8[:, None, :]).reshape(HID, HEADS)
    return jnp.pad(b, ((0, 0), (0, 8)))


def kernel(x_gene, x_disease, proj_gene_W, proj_gene_b, proj_disease_W,
           proj_disease_b, att_src_gd, att_dst_gd, att_src_dg, att_dst_dg,
           att_src_gg, att_dst_gg, q, k_lin_W, k_lin_b, lin_W, lin_b,
           ei_gd, ei_dg, ei_gg):
    del att_src_gd, att_dst_gd, ei_gd

    mg = jnp.concatenate(
        [_att_block(att_dst_dg), _att_block(att_src_gg), _att_block(att_dst_gg),
         jnp.zeros((HID, 16), jnp.float32)], axis=1)
    md = jnp.pad(_att_block(att_src_dg), ((0, 0), (0, 48)))

    hg, hd, ag, ad = _proj_call(
        x_gene, x_disease, proj_gene_W, proj_gene_b.reshape(1, HID),
        proj_disease_W, proj_disease_b.reshape(1, HID), mg, md)

    h_stack = jnp.stack([hd, hg])
    atab = jnp.pad(jnp.stack([
        jnp.concatenate([ad[:, 0:8], ag[:, 0:8]], axis=1),
        jnp.concatenate([ag[:, 16:24], ag[:, 32:40]], axis=1),
    ]), ((0, 0), (0, NPAD - N), (0, HID - 16)))
    zpad = jnp.zeros((EA - E,), jnp.int32)
    npad = jnp.full((EA - E,), N, jnp.int32)
    src_stack = jnp.concatenate(
        [ei_dg[0].astype(jnp.int32), zpad, ei_gg[0].astype(jnp.int32), zpad])
    dst_stack = jnp.concatenate(
        [ei_dg[1].astype(jnp.int32), npad, ei_gg[1].astype(jnp.int32), npad])

    acc, ssum = _sc_call(h_stack, atab, src_stack, dst_stack)

    rmat = jnp.where((jnp.arange(HID)[None, :] // 16) == jnp.arange(HID)[:, None],
                     1.0, 0.0).astype(jnp.float32)
    outn, tsum = _norm_call(acc, ssum, rmat, k_lin_W, k_lin_b.reshape(1, HID))

    score = (q[0][None, :] * (tsum / N)).sum(-1)
    attn = jax.nn.softmax(score, axis=0)

    return _fin_call(attn, outn, lin_W, lin_b.reshape(1, OUT))


# speedup vs baseline: 1.0364x; 1.0364x over previous
import jax
import jax.numpy as jnp
from jax import lax
from jax.experimental import pallas as pl
from jax.experimental.pallas import tpu as pltpu
from jax.experimental.pallas import tpu_sc as plsc

N = 10000
D_IN = 256
HID = 128
HEADS = 8
E = 160000
OUT = 128

BLK = 1000
NT = 16
CH = 64
NCHP = 157
EA = (NCHP + 1) * NT * CH
NPAD = N + 8
RCH = 40
NRCH = N // RCH


def _proj_body(xg_ref, xd_ref, wg_ref, bg_ref, wd_ref, bd_ref, mg_ref, md_ref,
               hg_ref, hd_ref, ag_ref, ad_ref):
    hg = jnp.dot(xg_ref[...], wg_ref[...], preferred_element_type=jnp.float32) + bg_ref[...]
    hd = jnp.dot(xd_ref[...], wd_ref[...], preferred_element_type=jnp.float32) + bd_ref[...]
    hg_ref[...] = hg
    hd_ref[...] = hd
    ag_ref[...] = jnp.dot(hg, mg_ref[...], preferred_element_type=jnp.float32)
    ad_ref[...] = jnp.dot(hd, md_ref[...], preferred_element_type=jnp.float32)


def _proj_call(xg, xd, wg, bg, wd, bd, mg, md):
    rb = lambda i: (i, 0)
    full = lambda i: (0, 0)
    return pl.pallas_call(
        _proj_body,
        grid=(N // BLK,),
        in_specs=[
            pl.BlockSpec((BLK, D_IN), rb),
            pl.BlockSpec((BLK, D_IN), rb),
            pl.BlockSpec((D_IN, HID), full),
            pl.BlockSpec((1, HID), full),
            pl.BlockSpec((D_IN, HID), full),
            pl.BlockSpec((1, HID), full),
            pl.BlockSpec((HID, 64), full),
            pl.BlockSpec((HID, 64), full),
        ],
        out_specs=[
            pl.BlockSpec((BLK, HID), rb),
            pl.BlockSpec((BLK, HID), rb),
            pl.BlockSpec((BLK, 64), rb),
            pl.BlockSpec((BLK, 64), rb),
        ],
        out_shape=[
            jax.ShapeDtypeStruct((N, HID), jnp.float32),
            jax.ShapeDtypeStruct((N, HID), jnp.float32),
            jax.ShapeDtypeStruct((N, 64), jnp.float32),
            jax.ShapeDtypeStruct((N, 64), jnp.float32),
        ],
    )(xg, xd, wg, bg, wd, bd, mg, md)


def _nchunks_t(t, total):
    return jnp.where(t < total - NT * (total // NT), total // NT + 1, total // NT)


def _sc_w_body(atab_hbm, src_hbm, dst_hbm,
               w_out_hbm, ssum_hbm,
               s_sh, idx2, idx_d, abrows, wpack, sbuf, zbuf,
               sem_a0, sem_a1, sem_b0, sem_b1, sem_g0, sem_g1):
    sem_a = (sem_a0, sem_a1)
    sem_b = (sem_b0, sem_b1)
    sem_g = (sem_g0, sem_g1)
    c = lax.axis_index("c")
    t = lax.axis_index("s")

    def zrow(i, _):
        for j in range(HID // 16):
            zbuf[i, pl.ds(16 * j, 16)] = jnp.zeros((16,), jnp.float32)
        return 0
    lax.fori_loop(0, RCH, zrow, 0)

    def zsrow(i, _):
        for j in range(HID // 16):
            sbuf[i, pl.ds(16 * j, 16)] = jnp.zeros((16,), jnp.float32)
        return 0
    lax.fori_loop(0, CH, zsrow, 0)

    def zchunk(k, _):
        r = (k * NT + t) * RCH
        pltpu.sync_copy(zbuf, s_sh.at[pl.ds(r, RCH)])
        return 0
    lax.fori_loop(0, _nchunks_t(t, NRCH), zchunk, 0)

    plsc.subcore_barrier()

    def start_idx(k, sl):
        b = c * EA + (k * NT + t) * CH
        pltpu.async_copy(src_hbm.at[pl.ds(b, CH)], idx2.at[sl, pl.ds(0, CH)],
                         sem_a[sl])
        pltpu.async_copy(dst_hbm.at[pl.ds(b, CH)], idx2.at[sl, pl.ds(CH, CH)],
                         sem_a[sl])
        pltpu.async_copy(dst_hbm.at[pl.ds(b, CH)], idx_d.at[sl], sem_b[sl])

    def start_gather(k, sl):
        b = c * EA + (k * NT + t) * CH
        pltpu.make_async_copy(src_hbm.at[pl.ds(b, CH)], idx2.at[sl, pl.ds(0, CH)],
                              sem_a[sl]).wait()
        pltpu.make_async_copy(dst_hbm.at[pl.ds(b, CH)], idx2.at[sl, pl.ds(CH, CH)],
                              sem_a[sl]).wait()
        pltpu.async_copy(atab_hbm.at[c].at[idx2.at[sl]], abrows.at[sl],
                         sem_g[sl])

    def process(k, sl):
        b = c * EA + (k * NT + t) * CH
        pltpu.make_async_copy(atab_hbm.at[c].at[idx2.at[sl]], abrows.at[sl],
                              sem_g[sl]).wait()
        pltpu.make_async_copy(dst_hbm.at[pl.ds(b, CH)], idx_d.at[sl],
                              sem_b[sl]).wait()

        @plsc.parallel_loop(0, CH, unroll=4)
        def wrow(i):
            a16 = abrows[sl, i, pl.ds(0, 16)]
            b16 = abrows[sl, CH + i, pl.ds(8, 16)]
            x = a16 + b16
            x = jnp.where(x >= 0.0, x, 0.2 * x)
            w = jnp.exp(x)
            sbuf[i, pl.ds(0, 16)] = w
            wpack[i // 8, pl.ds((i % 8) * 16, 16)] = w

        pltpu.sync_copy(sbuf, s_sh.at[idx_d.at[sl]], add=True)
        pltpu.sync_copy(wpack, w_out_hbm.at[pl.ds(pl.multiple_of(b // 8, 8), CH // 8)])

    start_idx(0, 0)
    start_gather(0, 0)
    start_idx(1, 1)

    def pair(kk, _):
        k0 = 2 * kk
        start_gather(k0 + 1, 1)
        process(k0, 0)
        start_idx(k0 + 2, 0)
        start_gather(k0 + 2, 0)
        process(k0 + 1, 1)
        start_idx(k0 + 3, 1)
        return 0
    lax.fori_loop(0, (NCHP - 1) // 2, pair, 0)
    process(NCHP - 1, 0)
    pltpu.make_async_copy(src_hbm.at[pl.ds(0, CH)], idx2.at[1, pl.ds(0, CH)],
                          sem_a[1]).wait()
    pltpu.make_async_copy(dst_hbm.at[pl.ds(0, CH)], idx2.at[1, pl.ds(CH, CH)],
                          sem_a[1]).wait()
    pltpu.make_async_copy(dst_hbm.at[pl.ds(0, CH)], idx_d.at[1], sem_b[1]).wait()

    plsc.subcore_barrier()

    def wb(k, _):
        r = (k * NT + t) * RCH
        pltpu.sync_copy(s_sh.at[pl.ds(r, RCH)], zbuf)
        pltpu.sync_copy(zbuf, ssum_hbm.at[c, pl.ds(r, RCH)])
        return 0
    lax.fori_loop(0, _nchunks_t(t, NRCH), wb, 0)


def _sc_m_body(h_hbm, w_hbm, src_hbm, dst_hbm,
               acc_hbm,
               acc_sh, idx_s, idx_d, wpack, hrows, zbuf,
               sem_a0, sem_a1, sem_b0, sem_b1, sem_g0, sem_g1):
    sem_a = (sem_a0, sem_a1)
    sem_b = (sem_b0, sem_b1)
    sem_g = (sem_g0, sem_g1)
    c = lax.axis_index("c")
    t = lax.axis_index("s")

    def zrow(i, _):
        for j in range(HID // 16):
            zbuf[i, pl.ds(16 * j, 16)] = jnp.zeros((16,), jnp.float32)
        return 0
    lax.fori_loop(0, RCH, zrow, 0)

    def zchunk(k, _):
        r = (k * NT + t) * RCH
        pltpu.sync_copy(zbuf, acc_sh.at[pl.ds(r, RCH)])
        return 0
    lax.fori_loop(0, _nchunks_t(t, NRCH), zchunk, 0)

    plsc.subcore_barrier()

    def start_idx(k, sl):
        b = c * EA + (k * NT + t) * CH
        pltpu.async_copy(src_hbm.at[pl.ds(b, CH)], idx_s.at[sl], sem_a[sl])
        pltpu.async_copy(dst_hbm.at[pl.ds(b, CH)], idx_d.at[sl], sem_b[sl])
        pltpu.async_copy(
            w_hbm.at[pl.ds(pl.multiple_of(b // 8, 8), CH // 8)], wpack.at[sl],
            sem_b[sl])

    def start_gather(k, sl):
        b = c * EA + (k * NT + t) * CH
        pltpu.make_async_copy(src_hbm.at[pl.ds(b, CH)], idx_s.at[sl],
                              sem_a[sl]).wait()
        pltpu.async_copy(h_hbm.at[c].at[idx_s.at[sl]], hrows.at[sl],
                         sem_g[sl])

    def process(k, sl):
        b = c * EA + (k * NT + t) * CH
        pltpu.make_async_copy(dst_hbm.at[pl.ds(b, CH)], idx_d.at[sl],
                              sem_b[sl]).wait()
        pltpu.make_async_copy(
            w_hbm.at[pl.ds(pl.multiple_of(b // 8, 8), CH // 8)], wpack.at[sl],
            sem_b[sl]).wait()
        pltpu.make_async_copy(h_hbm.at[c].at[idx_s.at[sl]], hrows.at[sl],
                              sem_g[sl]).wait()

        @plsc.parallel_loop(0, CH, unroll=4)
        def srow(i):
            wrow = wpack[sl, i // 8, pl.ds((i % 8) * 16, 16)]
            for h in range(HEADS):
                hrows[sl, i, pl.ds(16 * h, 16)] = (
                    hrows[sl, i, pl.ds(16 * h, 16)] * wrow[h])

        pltpu.sync_copy(hrows.at[sl], acc_sh.at[idx_d.at[sl]], add=True)

    start_idx(0, 0)
    start_gather(0, 0)
    start_idx(1, 1)

    def pair(kk, _):
        k0 = 2 * kk
        start_gather(k0 + 1, 1)
        process(k0, 0)
        start_idx(k0 + 2, 0)
        start_gather(k0 + 2, 0)
        process(k0 + 1, 1)
        start_idx(k0 + 3, 1)
        return 0
    lax.fori_loop(0, (NCHP - 1) // 2, pair, 0)
    process(NCHP - 1, 0)
    pltpu.make_async_copy(src_hbm.at[pl.ds(0, CH)], idx_s.at[1], sem_a[1]).wait()
    pltpu.make_async_copy(dst_hbm.at[pl.ds(0, CH)], idx_d.at[1], sem_b[1]).wait()
    pltpu.make_async_copy(w_hbm.at[pl.ds(0, CH // 8)], wpack.at[1], sem_b[1]).wait()

    plsc.subcore_barrier()

    def wb(k, _):
        r = (k * NT + t) * RCH
        pltpu.sync_copy(acc_sh.at[pl.ds(r, RCH)], zbuf)
        pltpu.sync_copy(zbuf, acc_hbm.at[c, pl.ds(r, RCH)])
        return 0
    lax.fori_loop(0, _nchunks_t(t, NRCH), wb, 0)


def _sc_call(h_stack, atab, src_stack, dst_stack):
    mesh = plsc.VectorSubcoreMesh(core_axis_name="c", subcore_axis_name="s")
    w_all, ssum = pl.kernel(
        _sc_w_body,
        out_type=[
            jax.ShapeDtypeStruct((2 * EA // 8, HID), jnp.float32),
            jax.ShapeDtypeStruct((2, N, HID), jnp.float32),
        ],
        mesh=mesh,
        scratch_types=[
            pltpu.VMEM_SHARED((NPAD, HID), jnp.float32),
            pltpu.VMEM((2, 2 * CH), jnp.int32),
            pltpu.VMEM((2, CH), jnp.int32),
            pltpu.VMEM((2, 2 * CH, HID), jnp.float32),
            pltpu.VMEM((CH // 8, HID), jnp.float32),
            pltpu.VMEM((CH, HID), jnp.float32),
            pltpu.VMEM((RCH, HID), jnp.float32),
        ] + [pltpu.SemaphoreType.DMA] * 6,
    )(atab, src_stack, dst_stack)

    acc = pl.kernel(
        _sc_m_body,
        out_type=jax.ShapeDtypeStruct((2, N, HID), jnp.float32),
        mesh=plsc.VectorSubcoreMesh(core_axis_name="c", subcore_axis_name="s"),
        scratch_types=[
            pltpu.VMEM_SHARED((NPAD, HID), jnp.float32),
            pltpu.VMEM((2, CH), jnp.int32),
            pltpu.VMEM((2, CH), jnp.int32),
            pltpu.VMEM((2, CH // 8, HID), jnp.float32),
            pltpu.VMEM((2, CH, HID), jnp.float32),
            pltpu.VMEM((RCH, HID), jnp.float32),
        ] + [pltpu.SemaphoreType.DMA] * 6,
    )(h_stack, w_all, src_stack, dst_stack)
    return acc, ssum


def _norm_body(acc_ref, s_ref, r_ref, kw_ref, kb_ref, outn_ref, tsum_ref):
    t = pl.program_id(0)
    i = pl.program_id(1)
    srep = jnp.dot(s_ref[0], r_ref[...], preferred_element_type=jnp.float32)
    a = acc_ref[0]
    o = jnp.where(srep > 0.0, a / srep, 0.0)
    o = jnp.maximum(o, 0.0)
    outn_ref[0] = o
    ts = jnp.sum(
        jnp.tanh(jnp.dot(o, kw_ref[...], preferred_element_type=jnp.float32) + kb_ref[...]),
        axis=0, keepdims=True)

    @pl.when((t == 0) & (i == 0))
    def _():
        tsum_ref[...] = jnp.zeros((2, HID), jnp.float32)

    row = lax.broadcasted_iota(jnp.int32, (2, HID), 0)
    tsum_ref[...] = tsum_ref[...] + jnp.where(row == t, ts, 0.0)


def _norm_call(acc, ssum, r, kw, kb):
    tb = lambda t, i: (t, i, 0)
    full = lambda t, i: (0, 0)
    return pl.pallas_call(
        _norm_body,
        grid=(2, N // BLK),
        in_specs=[
            pl.BlockSpec((1, BLK, HID), tb),
            pl.BlockSpec((1, BLK, HID), tb),
            pl.BlockSpec((HID, HID), full),
            pl.BlockSpec((HID, HID), full),
            pl.BlockSpec((1, HID), full),
        ],
        out_specs=[
            pl.BlockSpec((1, BLK, HID), tb),
            pl.BlockSpec((2, HID), lambda t, i: (0, 0)),
        ],
        out_shape=[
            jax.ShapeDtypeStruct((2, N, HID), jnp.float32),
            jax.ShapeDtypeStruct((2, HID), jnp.float32),
        ],
    )(acc, ssum, r, kw, kb)


def _fin_body(attn_ref, outn_ref, w_ref, b_ref, o_ref):
    g = attn_ref[0] * outn_ref[0] + attn_ref[1] * outn_ref[1]
    o_ref[...] = jnp.dot(g, w_ref[...], preferred_element_type=jnp.float32) + b_ref[...]


def _fin_call(attn, outn, w, b):
    return pl.pallas_call(
        _fin_body,
        grid=(N // BLK,),
        in_specs=[
            pl.BlockSpec(memory_space=pltpu.SMEM),
            pl.BlockSpec((2, BLK, HID), lambda i: (0, i, 0)),
            pl.BlockSpec((HID, OUT), lambda i: (0, 0)),
            pl.BlockSpec((1, OUT), lambda i: (0, 0)),
        ],
        out_specs=pl.BlockSpec((BLK, OUT), lambda i: (i, 0)),
        out_shape=jax.ShapeDtypeStruct((N, OUT), jnp.float32),
    )(attn, outn, w, b)


def _att_block(att):
    eye8 = jnp.eye(HEADS, dtype=jnp.float32)
    b = (att[0][:, :, None] * eye8[:, None, :]).reshape(HID, HEADS)
    return jnp.pad(b, ((0, 0), (0, 8)))


def kernel(x_gene, x_disease, proj_gene_W, proj_gene_b, proj_disease_W,
           proj_disease_b, att_src_gd, att_dst_gd, att_src_dg, att_dst_dg,
           att_src_gg, att_dst_gg, q, k_lin_W, k_lin_b, lin_W, lin_b,
           ei_gd, ei_dg, ei_gg):
    del att_src_gd, att_dst_gd, ei_gd

    mg = jnp.concatenate(
        [_att_block(att_dst_dg), _att_block(att_src_gg), _att_block(att_dst_gg),
         jnp.zeros((HID, 16), jnp.float32)], axis=1)
    md = jnp.pad(_att_block(att_src_dg), ((0, 0), (0, 48)))

    hg, hd, ag, ad = _proj_call(
        x_gene, x_disease, proj_gene_W, proj_gene_b.reshape(1, HID),
        proj_disease_W, proj_disease_b.reshape(1, HID), mg, md)

    h_stack = jnp.stack([hd, hg])
    atab = jnp.pad(jnp.stack([
        jnp.concatenate([ad[:, 0:8], ag[:, 0:8]], axis=1),
        jnp.concatenate([ag[:, 16:24], ag[:, 32:40]], axis=1),
    ]), ((0, 0), (0, NPAD - N), (0, HID - 16)))
    zpad = jnp.zeros((EA - E,), jnp.int32)
    npad = jnp.full((EA - E,), N, jnp.int32)
    src_stack = jnp.concatenate(
        [ei_dg[0].astype(jnp.int32), zpad, ei_gg[0].astype(jnp.int32), zpad])
    dst_stack = jnp.concatenate(
        [ei_dg[1].astype(jnp.int32), npad, ei_gg[1].astype(jnp.int32), npad])

    acc, ssum = _sc_call(h_stack, atab, src_stack, dst_stack)

    rmat = jnp.where((jnp.arange(HID)[None, :] // 16) == jnp.arange(HID)[:, None],
                     1.0, 0.0).astype(jnp.float32)
    outn, tsum = _norm_call(acc, ssum, rmat, k_lin_W, k_lin_b.reshape(1, HID))

    score = (q[0][None, :] * (tsum / N)).sum(-1)
    attn = jax.nn.softmax(score, axis=0)

    return _fin_call(attn, outn, lin_W, lin_b.reshape(1, OUT))
